# baseline (speedup 1.0000x reference)
import functools

import jax
import jax.numpy as jnp
import numpy as np
from jax import lax
from jax.experimental import pallas as pl
from jax.experimental.pallas import tpu as pltpu
from jax.experimental.pallas import tpu_sc as plsc

D = 256
NF = 19
N_ACTIVE = 12288
N_SIDE = 6144
N_TOTAL = N_ACTIVE + N_SIDE
KPAD = 128

_B = 512

_NC, _NS, _L = 2, 16, 16
_NW = _NC * _NS
_CW = N_TOTAL // _NW
_C = 16
_G = 3 * _C
_SUB = _CW // _C

_EQ_SEGS = ((3, 3, 18), (4, 7, 21), (5, 2, 28), (6, 2, 30), (7, 2, 32),
            (8, 8, 34), (9, 4, 42), (10, 2, 46), (11, 16, 48))
_SEL = np.zeros((NF, KPAD), np.float32)
_TGT = np.full((1, KPAD), -1.0, np.float32)
for _f, _n, _base in _EQ_SEGS:
    for _t in range(_n):
        _SEL[_f, _base + _t] = 1.0
        _TGT[0, _base + _t] = float(_t)


def _tc_body(feats_ref, g_ref, wcat_ref, bias_ref, sel_ref, tgt_ref, out_ref):
    feats = feats_ref[...]
    hp = feats[:, 0:1].astype(jnp.float32)
    maxhp = jnp.maximum(feats[:, 1:2], 1).astype(jnp.float32)
    ratio = jnp.clip(hp / maxhp, 0.0, 1.0)
    token = (1023.0 * ratio).astype(jnp.int32)
    lvl = feats[:, 2:3]
    m0 = feats[:, 15:16]
    m1 = feats[:, 16:17]
    m2 = feats[:, 17:18]

    c = lax.broadcasted_iota(jnp.int32, (_B, KPAD), 1)
    bitsrc = jnp.where(c < 10, token, lvl)
    sh = jnp.clip(jnp.where(c < 10, c, c - 11), 0, 31)
    bits = (lax.shift_right_logical(bitsrc, sh) & 1).astype(jnp.float32)
    fsel = jnp.dot(feats.astype(jnp.float32), sel_ref[...],
                   preferred_element_type=jnp.float32)
    eq = (jnp.abs(fsel - tgt_ref[...]) < 0.5).astype(jnp.float32)
    cm = c - 64
    mv = (((m0 == cm) | (m1 == cm) | (m2 == cm)) & (c < 68)).astype(jnp.float32)
    code = jnp.where(c == 10, ratio,
                     jnp.where(c < 18, bits,
                               jnp.where(c < 64, eq, mv)))
    out_ref[...] = (jnp.dot(code, wcat_ref[...],
                            preferred_element_type=jnp.float32)
                    + bias_ref[...] + g_ref[...].astype(jnp.float32))


def _tc_call(feats, g, wcat, bias, nrows, block_off):
    grid = nrows // _B
    return pl.pallas_call(
        _tc_body,
        grid=(grid,),
        in_specs=[
            pl.BlockSpec((_B, NF), lambda i: (i + block_off, 0)),
            pl.BlockSpec((_B, D), lambda i: (i + block_off, 0)),
            pl.BlockSpec((KPAD, D), lambda i: (0, 0)),
            pl.BlockSpec((1, D), lambda i: (0, 0)),
            pl.BlockSpec((NF, KPAD), lambda i: (0, 0)),
            pl.BlockSpec((1, KPAD), lambda i: (0, 0)),
        ],
        out_specs=pl.BlockSpec((_B, D), lambda i: (i, 0)),
        out_shape=jax.ShapeDtypeStruct((nrows, D), jnp.float32),
    )(feats, g, wcat, bias, jnp.asarray(_SEL), jnp.asarray(_TGT))


_TROWS = 3328


@functools.cache
def _sc_gather():
    mesh = plsc.VectorSubcoreMesh(core_axis_name="c", subcore_axis_name="s",
                                  num_cores=_NC)

    @functools.partial(
        pl.kernel,
        mesh=mesh,
        out_type=jax.ShapeDtypeStruct((N_TOTAL, D // 2), jnp.int32),
        scratch_types=[
            pltpu.VMEM((_SUB, _G), jnp.int32),
            pltpu.VMEM((_G, D // 2), jnp.int32),
            pltpu.VMEM((_G, D // 2), jnp.int32),
            pltpu.VMEM((_C, D // 2), jnp.int32),
            pltpu.VMEM((_C, D // 2), jnp.int32),
            pltpu.SemaphoreType.DMA,
            pltpu.SemaphoreType.DMA,
        ],
    )
    def sc_fn(table_hbm, idx_hbm, g_hbm, idxv, rA, rB, oA, oB,
              gsem, wsem):
        sid = lax.axis_index("s")
        w = sid * _NC + lax.axis_index("c")
        row0 = w * _CW
        pltpu.sync_copy(idx_hbm.at[w], idxv)
        rbufs = (rA, rB)
        obufs = (oA, oB)
        pltpu.async_copy(table_hbm.at[idxv.at[0]], rA, gsem)
        pltpu.async_copy(table_hbm.at[idxv.at[1]], rB, gsem)

        @pl.loop(0, _SUB, step=2)
        def chunk_loop(g):
            for b in range(2):
                k = g + b
                r = rbufs[b]
                o = obufs[b]
                pltpu.make_async_copy(table_hbm.at[idxv.at[0]], r, gsem).wait()
                @pl.when(g > 0)
                def _():
                    pltpu.make_async_copy(
                        o, g_hbm.at[pl.ds(row0, _C)], wsem).wait()

                @plsc.parallel_loop(0, _C)
                def add_rows(i):
                    msk = jnp.int32(-65536)
                    for j in range(D // (2 * _L)):
                        sl = pl.ds(j * _L, _L)
                        x0 = r[3 * i, sl]
                        x1 = r[3 * i + 1, sl]
                        x2 = r[3 * i + 2, sl]
                        def _f(v):
                            return lax.bitcast_convert_type(v, jnp.float32)
                        lo = (_f(x0 << 16) + _f(x1 << 16) + _f(x2 << 16))
                        hi = (_f(x0 & msk) + _f(x1 & msk) + _f(x2 & msk))
                        lo_i = lax.bitcast_convert_type(lo, jnp.int32)
                        hi_i = lax.bitcast_convert_type(hi, jnp.int32)
                        o[i, sl] = (lax.shift_right_logical(lo_i, 16)
                                    | (hi_i & msk))

                pltpu.async_copy(
                    o, g_hbm.at[pl.ds(row0 + k * _C, _C)], wsem)
                @pl.when(k + 2 < _SUB)
                def _():
                    pltpu.async_copy(table_hbm.at[idxv.at[k + 2]], r, gsem)

        pltpu.make_async_copy(oA, g_hbm.at[pl.ds(row0, _C)], wsem).wait()
        pltpu.make_async_copy(oB, g_hbm.at[pl.ds(row0, _C)], wsem).wait()

    return sc_fn


def kernel(active_entities, side_entities, W_onehot, b_onehot, W_species,
           b_species, W_ability, b_ability, W_item, b_item, W_moveset,
           b_moveset):
    feats = jnp.concatenate(
        [active_entities.reshape(N_ACTIVE, NF), side_entities], axis=0)
    wcat = jnp.concatenate(
        [W_onehot, W_moveset, jnp.zeros((KPAD - 68, D), jnp.float32)], axis=0)
    bias = (b_onehot + b_species + b_ability + b_item + b_moveset).reshape(1, D)
    table = lax.bitcast_convert_type(
        jnp.concatenate(
            [W_species, W_ability, jnp.zeros((1024 - 320, D), jnp.float32),
             W_item], axis=0).astype(jnp.bfloat16).reshape(_TROWS, D // 2, 2),
        jnp.int32)
    idx = jnp.stack([
        feats[:, 12],
        feats[:, 13] + 1280,
        feats[:, 14] + 2304,
    ], axis=1).astype(jnp.int32).reshape(_NW, _SUB, _G)
    g = lax.bitcast_convert_type(
        _sc_gather()(table, idx), jnp.bfloat16).reshape(N_TOTAL, D)
    out1 = _tc_call(feats, g, wcat, bias, N_ACTIVE, 0)
    out2 = _tc_call(feats, g, wcat, bias, N_SIDE, N_ACTIVE // _B)
    return out1.reshape(1024, 12, D), out2


# speedup vs baseline: 1.7598x; 1.7598x over previous
import functools

import jax
import jax.numpy as jnp
import numpy as np
from jax import lax
from jax.experimental import pallas as pl
from jax.experimental.pallas import tpu as pltpu
from jax.experimental.pallas import tpu_sc as plsc

D = 256
NF = 19
N_ACTIVE = 12288
N_SIDE = 6144
N_TOTAL = N_ACTIVE + N_SIDE
KPAD = 128

_B = 512

_NC, _NS, _L = 2, 16, 16
_NW = _NC * _NS
_CW = N_TOTAL // _NW
_C = 16
_G = 3 * _C
_SUB = _CW // _C

_EQ_SEGS = ((3, 3, 18), (4, 7, 21), (5, 2, 28), (6, 2, 30), (7, 2, 32),
            (8, 8, 34), (9, 4, 42), (10, 2, 46), (11, 16, 48))
_SEL = np.zeros((NF, KPAD), np.float32)
_TGT = np.full((1, KPAD), -1.0, np.float32)
for _f, _n, _base in _EQ_SEGS:
    for _t in range(_n):
        _SEL[_f, _base + _t] = 1.0
        _TGT[0, _base + _t] = float(_t)


def _tc_body(feats_ref, g_ref, wcat_ref, bias_ref, sel_ref, tgt_ref, out_ref):
    feats = feats_ref[...]
    hp = feats[:, 0:1].astype(jnp.float32)
    maxhp = jnp.maximum(feats[:, 1:2], 1).astype(jnp.float32)
    ratio = jnp.clip(hp / maxhp, 0.0, 1.0)
    token = (1023.0 * ratio).astype(jnp.int32)
    lvl = feats[:, 2:3]
    m0 = feats[:, 15:16]
    m1 = feats[:, 16:17]
    m2 = feats[:, 17:18]

    c = lax.broadcasted_iota(jnp.int32, (_B, KPAD), 1)
    bitsrc = jnp.where(c < 10, token, lvl)
    sh = jnp.clip(jnp.where(c < 10, c, c - 11), 0, 31)
    bits = (lax.shift_right_logical(bitsrc, sh) & 1).astype(jnp.float32)
    fsel = jnp.dot(feats.astype(jnp.float32), sel_ref[...],
                   preferred_element_type=jnp.float32)
    eq = (jnp.abs(fsel - tgt_ref[...]) < 0.5).astype(jnp.float32)
    cm = c - 64
    mv = (((m0 == cm) | (m1 == cm) | (m2 == cm)) & (c < 68)).astype(jnp.float32)
    code = jnp.where(c == 10, ratio,
                     jnp.where(c < 18, bits,
                               jnp.where(c < 64, eq, mv)))
    gi = g_ref[...]
    glo = lax.bitcast_convert_type(gi << 16, jnp.float32)
    ghi = lax.bitcast_convert_type(gi & jnp.int32(-65536), jnp.float32)
    out_ref[...] = (jnp.dot(code, wcat_ref[...],
                            preferred_element_type=jnp.float32)
                    + bias_ref[...]
                    + jnp.concatenate([glo, ghi], axis=1))


def _tc_call(feats, g, wcat, bias, nrows, block_off):
    grid = nrows // _B
    return pl.pallas_call(
        _tc_body,
        grid=(grid,),
        in_specs=[
            pl.BlockSpec((_B, NF), lambda i: (i + block_off, 0)),
            pl.BlockSpec((_B, D // 2), lambda i: (i + block_off, 0)),
            pl.BlockSpec((KPAD, D), lambda i: (0, 0)),
            pl.BlockSpec((1, D), lambda i: (0, 0)),
            pl.BlockSpec((NF, KPAD), lambda i: (0, 0)),
            pl.BlockSpec((1, KPAD), lambda i: (0, 0)),
        ],
        out_specs=pl.BlockSpec((_B, D), lambda i: (i, 0)),
        out_shape=jax.ShapeDtypeStruct((nrows, D), jnp.float32),
    )(feats, g, wcat, bias, jnp.asarray(_SEL), jnp.asarray(_TGT))


_TROWS = 2688


@functools.cache
def _sc_gather():
    mesh = plsc.VectorSubcoreMesh(core_axis_name="c", subcore_axis_name="s",
                                  num_cores=_NC)

    @functools.partial(
        pl.kernel,
        mesh=mesh,
        out_type=jax.ShapeDtypeStruct((N_TOTAL, D // 2), jnp.int32),
        scratch_types=[
            pltpu.VMEM((_SUB, _G), jnp.int32),
            pltpu.VMEM((_G, D // 2), jnp.int32),
            pltpu.VMEM((_G, D // 2), jnp.int32),
            pltpu.VMEM((_C, D // 2), jnp.int32),
            pltpu.VMEM((_C, D // 2), jnp.int32),
            pltpu.SemaphoreType.DMA,
            pltpu.SemaphoreType.DMA,
        ],
    )
    def sc_fn(table_hbm, idx_hbm, g_hbm, idxv, rA, rB, oA, oB,
              gsem, wsem):
        sid = lax.axis_index("s")
        w = sid * _NC + lax.axis_index("c")
        row0 = w * _CW
        pltpu.sync_copy(idx_hbm.at[w], idxv)
        rbufs = (rA, rB)
        obufs = (oA, oB)
        pltpu.async_copy(table_hbm.at[idxv.at[0]], rA, gsem)
        pltpu.async_copy(table_hbm.at[idxv.at[1]], rB, gsem)

        @pl.loop(0, _SUB, step=2)
        def chunk_loop(g):
            for b in range(2):
                k = g + b
                r = rbufs[b]
                o = obufs[b]
                pltpu.make_async_copy(table_hbm.at[idxv.at[0]], r, gsem).wait()
                @pl.when(g > 0)
                def _():
                    pltpu.make_async_copy(
                        o, g_hbm.at[pl.ds(row0, _C)], wsem).wait()

                @plsc.parallel_loop(0, _C)
                def add_rows(i):
                    msk = jnp.int32(-65536)
                    for j in range(D // (2 * _L)):
                        sl = pl.ds(j * _L, _L)
                        x0 = r[3 * i, sl]
                        x1 = r[3 * i + 1, sl]
                        x2 = r[3 * i + 2, sl]
                        def _f(v):
                            return lax.bitcast_convert_type(v, jnp.float32)
                        lo = (_f(x0 << 16) + _f(x1 << 16) + _f(x2 << 16))
                        hi = (_f(x0 & msk) + _f(x1 & msk) + _f(x2 & msk))
                        lo_i = lax.bitcast_convert_type(lo, jnp.int32)
                        hi_i = lax.bitcast_convert_type(hi, jnp.int32)
                        o[i, sl] = (lax.shift_right_logical(lo_i, 16)
                                    | (hi_i & msk))

                pltpu.async_copy(
                    o, g_hbm.at[pl.ds(row0 + k * _C, _C)], wsem)
                @pl.when(k + 2 < _SUB)
                def _():
                    pltpu.async_copy(table_hbm.at[idxv.at[k + 2]], r, gsem)

        pltpu.make_async_copy(oA, g_hbm.at[pl.ds(row0, _C)], wsem).wait()
        pltpu.make_async_copy(oB, g_hbm.at[pl.ds(row0, _C)], wsem).wait()

    return sc_fn


def kernel(active_entities, side_entities, W_onehot, b_onehot, W_species,
           b_species, W_ability, b_ability, W_item, b_item, W_moveset,
           b_moveset):
    feats = jnp.concatenate(
        [active_entities.reshape(N_ACTIVE, NF), side_entities], axis=0)
    wcat = jnp.concatenate(
        [W_onehot, W_moveset, jnp.zeros((KPAD - 68, D), jnp.float32)], axis=0)
    bias = (b_onehot + b_species + b_ability + b_item + b_moveset).reshape(1, D)
    tb = jnp.concatenate(
        [W_species, W_ability, jnp.zeros((64, D), jnp.float32),
         W_item], axis=0).astype(jnp.bfloat16)
    table = lax.bitcast_convert_type(
        jnp.stack([tb[:, :D // 2], tb[:, D // 2:]], axis=-1), jnp.int32)
    spec = feats[:, 12]
    abil = feats[:, 13]
    idx = jnp.stack([
        spec,
        jnp.where(abil < 320, abil + 1280, 1600 + (spec & 63)),
        feats[:, 14] + 1664,
    ], axis=1).astype(jnp.int32).reshape(_NW, _SUB, _G)
    g = _sc_gather()(table, idx)
    out1 = _tc_call(feats, g, wcat, bias, N_ACTIVE, 0)
    out2 = _tc_call(feats, g, wcat, bias, N_SIDE, N_ACTIVE // _B)
    return out1.reshape(1024, 12, D), out2


# speedup vs baseline: 1.8769x; 1.0666x over previous
import functools

import jax
import jax.numpy as jnp
import numpy as np
from jax import lax
from jax.experimental import pallas as pl
from jax.experimental.pallas import tpu as pltpu
from jax.experimental.pallas import tpu_sc as plsc

D = 256
NF = 19
N_ACTIVE = 12288
N_SIDE = 6144
N_TOTAL = N_ACTIVE + N_SIDE
KPAD = 128

_B = 512

_NC, _NS, _L = 2, 16, 16
_NW = _NC * _NS
_CW = N_TOTAL // _NW
_C = 16
_G = 3 * _C
_SUB = _CW // _C

_EQ_SEGS = ((3, 3, 18), (4, 7, 21), (5, 2, 28), (6, 2, 30), (7, 2, 32),
            (8, 8, 34), (9, 4, 42), (10, 2, 46), (11, 16, 48))
_SEL = np.zeros((NF, KPAD), np.float32)
_TGT = np.full((KPAD, 1), -1.0, np.float32)
for _f, _n, _base in _EQ_SEGS:
    for _t in range(_n):
        _SEL[_f, _base + _t] = 1.0
        _TGT[_base + _t, 0] = float(_t)
_SELT = np.ascontiguousarray(_SEL.T)


def _tc_body(featst_ref, g_ref, wcat_ref, bias_ref, selt_ref, tgt_ref,
             out_ref):
    feats = featst_ref[...]
    featsf = feats.astype(jnp.float32)
    fsel = jnp.dot(selt_ref[...], featsf,
                   preferred_element_type=jnp.float32)
    hp = featsf[0:1, :]
    maxhp = jnp.maximum(featsf[1:2, :], 1.0)
    ratio = jnp.clip(hp / maxhp, 0.0, 1.0)
    token = (1023.0 * ratio).astype(jnp.int32)
    lvl = feats[2:3, :]
    m0 = featsf[15:16, :]
    m1 = featsf[16:17, :]
    m2 = featsf[17:18, :]

    c = lax.broadcasted_iota(jnp.int32, (KPAD, _B), 0)
    bitsrc = jnp.where(c < 10, token, lvl)
    sh = jnp.clip(jnp.where(c < 10, c, c - 11), 0, 31)
    bits = (lax.shift_right_logical(bitsrc, sh) & 1).astype(jnp.float32)
    eq = (jnp.abs(fsel - tgt_ref[...]) < 0.5).astype(jnp.float32)
    cm = (c - 64).astype(jnp.float32)
    mv = (((m0 == cm) | (m1 == cm) | (m2 == cm)) & (c < 68)).astype(jnp.float32)
    code = jnp.where(c == 10, ratio,
                     jnp.where(c < 18, bits,
                               jnp.where(c < 64, eq, mv)))
    gi = g_ref[...]
    glo = lax.bitcast_convert_type(gi << 16, jnp.float32)
    ghi = lax.bitcast_convert_type(gi & jnp.int32(-65536), jnp.float32)
    out_ref[...] = (jnp.dot(jnp.transpose(code), wcat_ref[...],
                            preferred_element_type=jnp.float32)
                    + bias_ref[...]
                    + jnp.concatenate([glo, ghi], axis=1))


def _tc_call(featst, g, wcat, bias, nrows, block_off):
    grid = nrows // _B
    return pl.pallas_call(
        _tc_body,
        grid=(grid,),
        in_specs=[
            pl.BlockSpec((NF, _B), lambda i: (0, i + block_off)),
            pl.BlockSpec((_B, D // 2), lambda i: (i + block_off, 0)),
            pl.BlockSpec((KPAD, D), lambda i: (0, 0)),
            pl.BlockSpec((1, D), lambda i: (0, 0)),
            pl.BlockSpec((KPAD, NF), lambda i: (0, 0)),
            pl.BlockSpec((KPAD, 1), lambda i: (0, 0)),
        ],
        out_specs=pl.BlockSpec((_B, D), lambda i: (i, 0)),
        out_shape=jax.ShapeDtypeStruct((nrows, D), jnp.float32),
    )(featst, g, wcat, bias, jnp.asarray(_SELT), jnp.asarray(_TGT))


_TROWS = 2688


@functools.cache
def _sc_gather():
    mesh = plsc.VectorSubcoreMesh(core_axis_name="c", subcore_axis_name="s",
                                  num_cores=_NC)

    @functools.partial(
        pl.kernel,
        mesh=mesh,
        out_type=jax.ShapeDtypeStruct((N_TOTAL, D // 2), jnp.int32),
        scratch_types=[
            pltpu.VMEM((_SUB, _G), jnp.int32),
            pltpu.VMEM((_G, D // 2), jnp.int32),
            pltpu.VMEM((_G, D // 2), jnp.int32),
            pltpu.VMEM((_C, D // 2), jnp.int32),
            pltpu.VMEM((_C, D // 2), jnp.int32),
            pltpu.SemaphoreType.DMA,
            pltpu.SemaphoreType.DMA,
        ],
    )
    def sc_fn(table_hbm, idx_hbm, g_hbm, idxv, rA, rB, oA, oB,
              gsem, wsem):
        sid = lax.axis_index("s")
        w = sid * _NC + lax.axis_index("c")
        row0 = w * _CW
        pltpu.sync_copy(idx_hbm.at[w], idxv)
        rbufs = (rA, rB)
        obufs = (oA, oB)
        pltpu.async_copy(table_hbm.at[idxv.at[0]], rA, gsem)
        pltpu.async_copy(table_hbm.at[idxv.at[1]], rB, gsem)

        @pl.loop(0, _SUB, step=2)
        def chunk_loop(g):
            for b in range(2):
                k = g + b
                r = rbufs[b]
                o = obufs[b]
                pltpu.make_async_copy(table_hbm.at[idxv.at[0]], r, gsem).wait()
                @pl.when(g > 0)
                def _():
                    pltpu.make_async_copy(
                        o, g_hbm.at[pl.ds(row0, _C)], wsem).wait()

                @plsc.parallel_loop(0, _C)
                def add_rows(i):
                    msk = jnp.int32(-65536)
                    for j in range(D // (2 * _L)):
                        sl = pl.ds(j * _L, _L)
                        x0 = r[3 * i, sl]
                        x1 = r[3 * i + 1, sl]
                        x2 = r[3 * i + 2, sl]
                        def _f(v):
                            return lax.bitcast_convert_type(v, jnp.float32)
                        lo = (_f(x0 << 16) + _f(x1 << 16) + _f(x2 << 16))
                        hi = (_f(x0 & msk) + _f(x1 & msk) + _f(x2 & msk))
                        lo_i = lax.bitcast_convert_type(lo, jnp.int32)
                        hi_i = lax.bitcast_convert_type(hi, jnp.int32)
                        o[i, sl] = (lax.shift_right_logical(lo_i, 16)
                                    | (hi_i & msk))

                pltpu.async_copy(
                    o, g_hbm.at[pl.ds(row0 + k * _C, _C)], wsem)
                @pl.when(k + 2 < _SUB)
                def _():
                    pltpu.async_copy(table_hbm.at[idxv.at[k + 2]], r, gsem)

        pltpu.make_async_copy(oA, g_hbm.at[pl.ds(row0, _C)], wsem).wait()
        pltpu.make_async_copy(oB, g_hbm.at[pl.ds(row0, _C)], wsem).wait()

    return sc_fn


def kernel(active_entities, side_entities, W_onehot, b_onehot, W_species,
           b_species, W_ability, b_ability, W_item, b_item, W_moveset,
           b_moveset):
    featst = jnp.concatenate(
        [active_entities.reshape(N_ACTIVE, NF).T, side_entities.T], axis=1)
    wcat = jnp.concatenate(
        [W_onehot, W_moveset, jnp.zeros((KPAD - 68, D), jnp.float32)], axis=0)
    bias = (b_onehot + b_species + b_ability + b_item + b_moveset).reshape(1, D)
    tb = jnp.concatenate(
        [W_species, W_ability, jnp.zeros((64, D), jnp.float32),
         W_item], axis=0).astype(jnp.bfloat16)
    table = lax.bitcast_convert_type(
        jnp.stack([tb[:, :D // 2], tb[:, D // 2:]], axis=-1), jnp.int32)
    spec = featst[12, :]
    abil = featst[13, :]
    idx = jnp.stack([
        spec,
        jnp.where(abil < 320, abil + 1280, 1600 + (spec & 63)),
        featst[14, :] + 1664,
    ], axis=1).astype(jnp.int32).reshape(_NW, _SUB, _G)
    g = _sc_gather()(table, idx)
    out1 = _tc_call(featst, g, wcat, bias, N_ACTIVE, 0)
    out2 = _tc_call(featst, g, wcat, bias, N_SIDE, N_ACTIVE // _B)
    return out1.reshape(1024, 12, D), out2
